# Initial kernel scaffold; baseline (speedup 1.0000x reference)
import jax
import jax.numpy as jnp
from jax.experimental import pallas as pl


def kernel(r_idx, e1_idx, e2_idx, e3_idx, e4_idx, e5_idx, e6_idx, E_w, R_w, R_bias_w, Rw0, Rw1, Rw2, Rw3, Rw4, Rw5):
    raise NotImplementedError("write your pallas kernel here")


# speedup vs baseline: 8.6879x; 8.6879x over previous
import functools

import jax
import jax.numpy as jnp
from jax import lax
from jax.experimental import pallas as pl
from jax.experimental.pallas import tpu as pltpu
from jax.experimental.pallas import tpu_sc as plsc

_NUM_ENT = 1000000
_EMB = 64
_ARITY = 6
_BATCH = 4096
_RELW = 400

_NC, _NS = 2, 16
_NW = _NC * _NS


def _sc_gather(E_w, relT, eidx_all, r_idx):
    n_e = eidx_all.shape[0]
    n_r = r_idx.shape[0]
    epw = n_e // _NW
    rpw = n_r // _NW
    mesh = plsc.VectorSubcoreMesh(core_axis_name="c", subcore_axis_name="s")

    @functools.partial(
        pl.kernel,
        mesh=mesh,
        out_type=(
            jax.ShapeDtypeStruct((n_e, _EMB), jnp.float32),
            jax.ShapeDtypeStruct((n_r, _RELW), jnp.float32),
        ),
        scratch_types=[
            pltpu.VMEM((epw,), jnp.int32),
            pltpu.VMEM((epw, _EMB), jnp.float32),
            pltpu.VMEM((rpw,), jnp.int32),
            pltpu.VMEM((rpw, _RELW), jnp.float32),
            pltpu.SemaphoreType.DMA,
            pltpu.SemaphoreType.DMA,
        ],
        compiler_params=pltpu.CompilerParams(use_tc_tiling_on_sc=False),
    )
    def k(E_hbm, relT_hbm, eidx_hbm, ridx_hbm, emb_out, rel_out,
          eidx_v, emb_v, ridx_v, rel_v, sem_e, sem_r):
        wid = lax.axis_index("s") * _NC + lax.axis_index("c")
        be = wid * epw
        br = wid * rpw
        pltpu.sync_copy(eidx_hbm.at[pl.ds(be, epw)], eidx_v)
        pltpu.sync_copy(ridx_hbm.at[pl.ds(br, rpw)], ridx_v)
        cp_e = pltpu.async_copy(E_hbm.at[eidx_v], emb_v, sem_e)
        cp_r = pltpu.async_copy(relT_hbm.at[ridx_v], rel_v, sem_r)
        cp_e.wait()
        cp_r.wait()
        pltpu.sync_copy(emb_v, emb_out.at[pl.ds(be, epw)])
        pltpu.sync_copy(rel_v, rel_out.at[pl.ds(br, rpw)])

    return k(E_w, relT, eidx_all, r_idx)


def _tc_body(emb_ref, rel_ref, out_ref):
    emb = emb_ref[...]
    rel = rel_ref[...]
    prod = emb * rel[:, : _ARITY * _EMB]
    pa = []
    for a in range(_ARITY):
        acc = prod[:, a * 64 : a * 64 + 8]
        for w in range(1, 8):
            acc = acc + prod[:, a * 64 + w * 8 : a * 64 + w * 8 + 8]
        pa.append(acc)
    P = [pa[0]]
    for a in range(1, _ARITY):
        P.append(P[-1] + pa[a])
    S = [pa[5]]
    for a in range(4, 0, -1):
        S.append(S[-1] + pa[a])
    S = S[::-1]
    rb = rel[:, 384:392]
    variants = [P[0], S[0], P[1], S[1], P[2], S[2], P[3], S[3], P[4], S[4], P[5]]
    s = [jnp.sum(jnp.tanh(v + rb), axis=1) for v in variants]
    out = (rel[:, 392] * s[0] * s[1]
           + rel[:, 393] * s[2] * s[3]
           + rel[:, 394] * s[4] * s[5]
           + rel[:, 395] * s[6] * s[7]
           + rel[:, 396] * s[8] * s[9]
           + rel[:, 397] * s[10])
    out_ref[...] = out


def _tc_compute(emb, rel):
    B = rel.shape[0]
    BT = 512
    grid = (B // BT,)
    out = pl.pallas_call(
        _tc_body,
        grid=grid,
        in_specs=[
            pl.BlockSpec((BT, _ARITY * _EMB), lambda i: (i, 0)),
            pl.BlockSpec((BT, _RELW), lambda i: (i, 0)),
        ],
        out_specs=pl.BlockSpec((BT,), lambda i: (i,)),
        out_shape=jax.ShapeDtypeStruct((B,), jnp.float32),
    )(emb, rel)
    return out


def kernel(r_idx, e1_idx, e2_idx, e3_idx, e4_idx, e5_idx, e6_idx,
           E_w, R_w, R_bias_w, Rw0, Rw1, Rw2, Rw3, Rw4, Rw5):
    B = r_idx.shape[0]
    relT = jnp.concatenate(
        [R_w, R_bias_w, Rw0, Rw1, Rw2, Rw3, Rw4, Rw5,
         jnp.zeros((R_w.shape[0], 2), jnp.float32)], axis=1)
    eidx_all = jnp.stack(
        [e1_idx, e2_idx, e3_idx, e4_idx, e5_idx, e6_idx], axis=1
    ).reshape(-1).astype(jnp.int32)
    emb, rel = _sc_gather(E_w, relT, eidx_all, r_idx.astype(jnp.int32))
    return _tc_compute(emb.reshape(B, _ARITY * _EMB), rel)
